# baseline (speedup 1.0000x reference)
import functools

import jax
import jax.numpy as jnp
from jax import lax
from jax.experimental import pallas as pl
from jax.experimental.pallas import tpu as pltpu
from jax.experimental.pallas import tpu_sc as plsc

NUM_EMB = 1_000_000
DIM = 32
BATCH = 16384
HIST = 50
HP = 56
ESUP = NUM_EMB // 4
SUPW = 128

NC = 2
NS = 16
NW = NC * NS
CPW = (BATCH // 128) // NW
NSUB = CPW * HIST

SLABW = 512
NSLAB = 1952
COLS_PW = NSLAB // NW

_mesh = plsc.VectorSubcoreMesh(core_axis_name="c", subcore_axis_name="s")


@functools.partial(
    pl.kernel,
    out_type=jax.ShapeDtypeStruct((ESUP, SUPW), jnp.float32),
    mesh=_mesh,
    scratch_types=[
        pltpu.VMEM((DIM, SLABW), jnp.float32),
        pltpu.VMEM((DIM, SLABW), jnp.float32),
        pltpu.VMEM((SLABW // 4, SUPW), jnp.float32),
        pltpu.VMEM((SLABW // 4, SUPW), jnp.float32),
        pltpu.SemaphoreType.DMA,
        pltpu.SemaphoreType.DMA,
        pltpu.SemaphoreType.DMA,
        pltpu.SemaphoreType.DMA,
    ],
    compiler_params=pltpu.CompilerParams(needs_layout_passes=False),
)
def _detile_kernel(wt_hbm, wtail_hbm, out_hbm, slab0, slab1, ob0, ob1,
                   s_in0, s_in1, s_out0, s_out1):
    slab = (slab0, slab1)
    ob = (ob0, ob1)
    s_in = (s_in0, s_in1)
    s_out = (s_out0, s_out1)

    wid = lax.axis_index("s") * NC + lax.axis_index("c")
    iota16 = lax.iota(jnp.int32, 16)

    def col_of(n):
        return wid + NW * n

    def fetch_descr(n, p):
        c0 = pl.multiple_of(col_of(n) * SLABW, 128)
        return (wt_hbm.at[:, pl.ds(c0, SLABW)], slab[p], s_in[p])

    def write_descr(n, p):
        q0 = pl.multiple_of(col_of(n) * (SLABW // 4), 8)
        return (ob[p], out_hbm.at[pl.ds(q0, SLABW // 4), :], s_out[p])

    def extract(p):
        zero16 = jnp.zeros((16,), jnp.int32)

        @plsc.parallel_loop(0, (SLABW // 4) * 8, step=1, unroll=8)
        def _ex(n2):
            i = n2 // 8
            k = n2 % 8
            rows = (k % 2) * 16 + iota16
            col = zero16 + (4 * i + k // 2)
            g = plsc.load_gather(slab[p], [rows, col])
            ob[p][i, pl.ds(k * 16, 16)] = g

    pltpu.async_copy(*fetch_descr(0, 0))
    pltpu.async_copy(*fetch_descr(1, 1))

    pltpu.make_async_copy(*fetch_descr(0, 0)).wait()
    extract(0)
    pltpu.async_copy(*write_descr(0, 0))
    pltpu.async_copy(*fetch_descr(2, 0))

    def body(t, carry):
        for q in range(2):
            n = 1 + 2 * t + q
            p = (q + 1) % 2
            pltpu.make_async_copy(*fetch_descr(n, p)).wait()

            @pl.when(n >= 2)
            def _wait_write():
                pltpu.make_async_copy(*write_descr(n, p)).wait()

            extract(p)
            pltpu.async_copy(*write_descr(n, p))

            @pl.when(n < COLS_PW - 2)
            def _prefetch():
                pltpu.async_copy(*fetch_descr(n + 2, p))
        return carry

    lax.fori_loop(0, (COLS_PW - 1) // 2, body, 0)
    pltpu.make_async_copy(*write_descr(0, 0)).wait()
    pltpu.make_async_copy(*write_descr(0, 1)).wait()

    @pl.when(wid == 0)
    def _tail_full():
        c0 = pl.multiple_of(NSLAB * SLABW, 128)
        pltpu.sync_copy(wt_hbm.at[:, pl.ds(c0, SLABW)], slab0)
        extract(0)
        pltpu.sync_copy(ob0, out_hbm.at[pl.ds(NSLAB * (SLABW // 4),
                                              SLABW // 4), :])

    @pl.when(wid == 4)
    def _tail_partial():
        pltpu.sync_copy(wtail_hbm, ob1.at[pl.ds(0, 16), :])
        pltpu.sync_copy(ob1.at[pl.ds(0, 16), :],
                        out_hbm.at[pl.ds((NUM_EMB - 64) // 4, 16), :])


@functools.partial(
    pl.kernel,
    out_type=jax.ShapeDtypeStruct((HIST, DIM, BATCH), jnp.float32),
    mesh=_mesh,
    scratch_types=[
        pltpu.VMEM((8, 128), jnp.int32),
        pltpu.VMEM((128,), jnp.int32),
        pltpu.VMEM((128,), jnp.int32),
        pltpu.VMEM((128,), jnp.int32),
        pltpu.VMEM((128,), jnp.int32),
        pltpu.VMEM((128, SUPW), jnp.float32),
        pltpu.VMEM((128, SUPW), jnp.float32),
        pltpu.VMEM((DIM, 128), jnp.float32),
        pltpu.VMEM((DIM, 128), jnp.float32),
        pltpu.SemaphoreType.DMA,
        pltpu.SemaphoreType.DMA,
        pltpu.SemaphoreType.DMA,
        pltpu.SemaphoreType.DMA,
        pltpu.SemaphoreType.DMA,
    ],
    compiler_params=pltpu.CompilerParams(needs_layout_passes=False),
)
def _gather_kernel(wsup_hbm, idx_hbm, out_hbm,
                   idx_t, sup_idx0, sup_idx1, ext_b0, ext_b1,
                   sup_rows0, sup_rows1, trans0, trans1,
                   s_idx, s_gat0, s_gat1, s_out0, s_out1):
    sup_idx = (sup_idx0, sup_idx1)
    ext_b = (ext_b0, ext_b1)
    sup_rows = (sup_rows0, sup_rows1)
    trans = (trans0, trans1)
    s_gat = (s_gat0, s_gat1)
    s_out = (s_out0, s_out1)

    wid = lax.axis_index("s") * NC + lax.axis_index("c")
    iota16 = lax.iota(jnp.int32, 16)

    def idx_fetch_descr(s):
        c = s // HIST
        r = s % HIST
        b0 = (wid * CPW + c) * 128
        return (idx_hbm.at[pl.ds(pl.multiple_of(r - r % 8, 8), 8),
                           pl.ds(b0, 128)], idx_t, s_idx)

    def stage(s, p):
        r = s % HIST
        j = r % 8

        @pl.when(j == 0)
        def _wait_tile():
            pltpu.make_async_copy(*idx_fetch_descr(s)).wait()

        for j2 in range(8):
            v = idx_t[j, pl.ds(j2 * 16, 16)]
            sup_idx[p][pl.ds(j2 * 16, 16)] = v >> 2
            ext_b[p][pl.ds(j2 * 16, 16)] = (v & 3) * 32
        pltpu.async_copy(wsup_hbm.at[sup_idx[p]], sup_rows[p], s_gat[p])

        @pl.when(jnp.logical_and((s + 1) % HIST % 8 == 0, s < NSUB - 1))
        def _prefetch_tile():
            pltpu.async_copy(*idx_fetch_descr(s + 1))

    def drain(s, p, wait_write):
        c = s // HIST
        r = s % HIST
        b0 = (wid * CPW + c) * 128
        pltpu.make_async_copy(wsup_hbm.at[sup_idx[p]], sup_rows[p],
                              s_gat[p]).wait()
        if wait_write:
            pltpu.make_async_copy(
                trans[p], out_hbm.at[0, :, pl.ds(0, 128)], s_out[p]).wait()
        for j2 in range(8):
            ext = ext_b[p][pl.ds(j2 * 16, 16)]
            row = j2 * 16 + iota16

            @plsc.parallel_loop(0, DIM, step=1, unroll=8)
            def _ex(d):
                g = plsc.load_gather(sup_rows[p], [row, ext + d])
                trans[p][d, pl.ds(j2 * 16, 16)] = g
        pltpu.async_copy(trans[p], out_hbm.at[r, :, pl.ds(b0, 128)], s_out[p])

    pltpu.async_copy(*idx_fetch_descr(0))
    stage(0, 0)
    stage(1, 1)
    drain(0, 0, False)
    stage(2, 0)
    drain(1, 1, False)

    def body(t, carry):
        s = 3 + 2 * t
        stage(s, 1)
        drain(s - 1, 0, True)
        stage(s + 1, 0)
        drain(s, 1, True)
        return carry

    lax.fori_loop(0, (NSUB - 4) // 2, body, 0)

    stage(NSUB - 1, 1)
    drain(NSUB - 2, 0, True)
    drain(NSUB - 1, 1, True)
    pltpu.make_async_copy(trans0, out_hbm.at[0, :, pl.ds(0, 128)], s_out0).wait()
    pltpu.make_async_copy(trans1, out_hbm.at[0, :, pl.ds(0, 128)], s_out1).wait()


def kernel(input, weight):
    idxp = jnp.pad(input.T, ((0, HP - HIST), (0, 0)))
    wtail = weight[NUM_EMB - 64:].reshape(16, 128)
    wsup = _detile_kernel(weight.T, wtail)
    outk = _gather_kernel(wsup, idxp)
    return outk.transpose(2, 0, 1)


# speedup vs baseline: 1.2414x; 1.2414x over previous
import functools

import jax
import jax.numpy as jnp
from jax import lax
from jax.experimental import pallas as pl
from jax.experimental.pallas import tpu as pltpu
from jax.experimental.pallas import tpu_sc as plsc

NUM_EMB = 1_000_000
DIM = 32
BATCH = 16384
HIST = 50
HP = 56
ESUP = NUM_EMB // 4
SUPW = 128

NC = 2
NS = 16
NW = NC * NS
CPW = (BATCH // 128) // NW
NSUB = CPW * HIST

NCOL = NUM_EMB // 128
NFULL = 7808
COLS_PW = NFULL // NW

_mesh = plsc.VectorSubcoreMesh(core_axis_name="c", subcore_axis_name="s")


@functools.partial(
    pl.kernel,
    out_type=jax.ShapeDtypeStruct((ESUP, SUPW), jnp.float32),
    mesh=_mesh,
    scratch_types=[
        pltpu.VMEM((DIM, 128), jnp.float32),
        pltpu.VMEM((DIM, 128), jnp.float32),
        pltpu.VMEM((DIM, 128), jnp.float32),
        pltpu.VMEM((DIM, 128), jnp.float32),
        pltpu.SemaphoreType.DMA,
        pltpu.SemaphoreType.DMA,
        pltpu.SemaphoreType.DMA,
        pltpu.SemaphoreType.DMA,
    ],
    compiler_params=pltpu.CompilerParams(needs_layout_passes=False),
)
def _detile_kernel(wt_hbm, wtail_hbm, out_hbm, slab0, slab1, ob0, ob1,
                   s_in0, s_in1, s_out0, s_out1):
    slab = (slab0, slab1)
    ob = (ob0, ob1)
    s_in = (s_in0, s_in1)
    s_out = (s_out0, s_out1)

    wid = lax.axis_index("s") * NC + lax.axis_index("c")
    iota16 = lax.iota(jnp.int32, 16)
    rows16 = (iota16, iota16 + 16)

    def col_of(n):
        return wid + NW * n

    def fetch_descr(n, p):
        c0 = pl.multiple_of(col_of(n) * 128, 128)
        return (wt_hbm.at[:, pl.ds(c0, 128)], slab[p], s_in[p])

    def write_descr(n, p):
        q0 = pl.multiple_of(col_of(n) * DIM, 32)
        return (ob[p], out_hbm.at[pl.ds(q0, DIM), :], s_out[p])

    def extract(p, ni):
        zero16 = jnp.zeros((16,), jnp.int32)

        @plsc.parallel_loop(0, ni * 8, step=1, unroll=8)
        def _ex(n2):
            i = n2 // 8
            k = n2 % 8
            rows = (k % 2) * 16 + iota16
            col = zero16 + (4 * i + k // 2)
            g = plsc.load_gather(slab[p], [rows, col])
            ob[p][i, pl.ds(k * 16, 16)] = g

    pltpu.async_copy(*fetch_descr(0, 0))
    pltpu.async_copy(*fetch_descr(1, 1))

    def body(t, carry):
        for q in range(2):
            n = 2 * t + q
            pltpu.make_async_copy(*fetch_descr(n, q)).wait()

            @pl.when(n >= 2)
            def _wait_write():
                pltpu.make_async_copy(*write_descr(n, q)).wait()

            extract(q, DIM)
            pltpu.async_copy(*write_descr(n, q))

            @pl.when(n < COLS_PW - 2)
            def _prefetch():
                pltpu.async_copy(*fetch_descr(n + 2, q))
        return carry

    lax.fori_loop(0, COLS_PW // 2, body, 0)
    pltpu.make_async_copy(*write_descr(0, 0)).wait()
    pltpu.make_async_copy(*write_descr(0, 1)).wait()

    @pl.when(wid < 4)
    def _tail_full():
        c0 = pl.multiple_of((NFULL + wid) * 128, 128)
        pltpu.sync_copy(wt_hbm.at[:, pl.ds(c0, 128)], slab0)
        extract(0, DIM)
        pltpu.sync_copy(ob0, out_hbm.at[pl.ds((NFULL + wid) * DIM, DIM), :])

    @pl.when(wid == 4)
    def _tail_partial():
        pltpu.sync_copy(wtail_hbm, ob1.at[pl.ds(0, 16), :])
        pltpu.sync_copy(ob1.at[pl.ds(0, 16), :],
                        out_hbm.at[pl.ds(NCOL * DIM, 16), :])


@functools.partial(
    pl.kernel,
    out_type=jax.ShapeDtypeStruct((HIST, DIM, BATCH), jnp.float32),
    mesh=_mesh,
    scratch_types=[
        pltpu.VMEM((8, 128), jnp.int32),
        pltpu.VMEM((128,), jnp.int32),
        pltpu.VMEM((128,), jnp.int32),
        pltpu.VMEM((128,), jnp.int32),
        pltpu.VMEM((128,), jnp.int32),
        pltpu.VMEM((128, SUPW), jnp.float32),
        pltpu.VMEM((128, SUPW), jnp.float32),
        pltpu.VMEM((DIM, 128), jnp.float32),
        pltpu.VMEM((DIM, 128), jnp.float32),
        pltpu.SemaphoreType.DMA,
        pltpu.SemaphoreType.DMA,
        pltpu.SemaphoreType.DMA,
        pltpu.SemaphoreType.DMA,
        pltpu.SemaphoreType.DMA,
    ],
    compiler_params=pltpu.CompilerParams(needs_layout_passes=False),
)
def _gather_kernel(wsup_hbm, idx_hbm, out_hbm,
                   idx_t, sup_idx0, sup_idx1, ext_b0, ext_b1,
                   sup_rows0, sup_rows1, trans0, trans1,
                   s_idx, s_gat0, s_gat1, s_out0, s_out1):
    sup_idx = (sup_idx0, sup_idx1)
    ext_b = (ext_b0, ext_b1)
    sup_rows = (sup_rows0, sup_rows1)
    trans = (trans0, trans1)
    s_gat = (s_gat0, s_gat1)
    s_out = (s_out0, s_out1)

    wid = lax.axis_index("s") * NC + lax.axis_index("c")
    iota16 = lax.iota(jnp.int32, 16)

    def idx_fetch_descr(s):
        c = s // HIST
        r = s % HIST
        b0 = (wid * CPW + c) * 128
        return (idx_hbm.at[pl.ds(pl.multiple_of(r - r % 8, 8), 8),
                           pl.ds(b0, 128)], idx_t, s_idx)

    def stage(s, p):
        r = s % HIST
        j = r % 8

        @pl.when(j == 0)
        def _wait_tile():
            pltpu.make_async_copy(*idx_fetch_descr(s)).wait()

        for j2 in range(8):
            v = idx_t[j, pl.ds(j2 * 16, 16)]
            sup_idx[p][pl.ds(j2 * 16, 16)] = v >> 2
            ext_b[p][pl.ds(j2 * 16, 16)] = (v & 3) * 32
        pltpu.async_copy(wsup_hbm.at[sup_idx[p]], sup_rows[p], s_gat[p])

        @pl.when(jnp.logical_and((s + 1) % HIST % 8 == 0, s < NSUB - 1))
        def _prefetch_tile():
            pltpu.async_copy(*idx_fetch_descr(s + 1))

    def drain(s, p, wait_write):
        c = s // HIST
        r = s % HIST
        b0 = (wid * CPW + c) * 128
        pltpu.make_async_copy(wsup_hbm.at[sup_idx[p]], sup_rows[p],
                              s_gat[p]).wait()
        if wait_write:
            pltpu.make_async_copy(
                trans[p], out_hbm.at[0, :, pl.ds(0, 128)], s_out[p]).wait()
        for j2 in range(8):
            ext = ext_b[p][pl.ds(j2 * 16, 16)]
            row = j2 * 16 + iota16

            @plsc.parallel_loop(0, DIM, step=1, unroll=8)
            def _ex(t):
                civ = ((iota16 + t) & 15) + (t & 16)
                g = plsc.load_gather(sup_rows[p], [row, ext + civ])
                plsc.store_scatter(trans[p], [civ, row], g)
        pltpu.async_copy(trans[p], out_hbm.at[r, :, pl.ds(b0, 128)], s_out[p])

    pltpu.async_copy(*idx_fetch_descr(0))
    stage(0, 0)
    stage(1, 1)
    drain(0, 0, False)
    stage(2, 0)
    drain(1, 1, False)

    def body(t, carry):
        s = 3 + 2 * t
        stage(s, 1)
        drain(s - 1, 0, True)
        stage(s + 1, 0)
        drain(s, 1, True)
        return carry

    lax.fori_loop(0, (NSUB - 4) // 2, body, 0)

    stage(NSUB - 1, 1)
    drain(NSUB - 2, 0, True)
    drain(NSUB - 1, 1, True)
    pltpu.make_async_copy(trans0, out_hbm.at[0, :, pl.ds(0, 128)], s_out0).wait()
    pltpu.make_async_copy(trans1, out_hbm.at[0, :, pl.ds(0, 128)], s_out1).wait()


def kernel(input, weight):
    idxp = jnp.pad(input.T, ((0, HP - HIST), (0, 0)))
    wtail = weight[NUM_EMB - 64:].reshape(16, 128)
    wsup = _detile_kernel(weight.T, wtail)
    outk = _gather_kernel(wsup, idxp)
    return outk.transpose(2, 0, 1)
